# baseline (speedup 1.0000x reference)
import dataclasses
import functools

import jax
import jax.numpy as jnp
from jax import lax
from jax.experimental import pallas as pl
from jax.experimental.pallas import tpu as pltpu
from jax.experimental.pallas import tpu_sc as plsc

T = 16384
D_MODEL = 2048
N_EXPERTS = 64
N_ACTIVATED = 8
N_GROUPS = 8
GROUP_SIZE = N_EXPERTS // N_GROUPS
TOPK_GROUPS = 4
ROUTE_SCALE = 2.5

BT = 512

NEG_INF = float("-inf")

NC = 2
NS = 16
NW = NC * NS
LANES = 16

N_CHUNKS = 2
TCHUNK = T // N_CHUNKS
TOK_PER_TILE = TCHUNK // NW
CHUNKS_PER_TILE = TOK_PER_TILE // LANES


def _tc_scores_block(x_ref, w_ref, bias_ref, scores_out_ref, routing_t_ref):
    x = x_ref[...]
    w = w_ref[...]
    dn = (((1,), (1,)), ((), ()))
    logits = lax.dot_general(x, w, dn, preferred_element_type=jnp.float32)
    scores_out_ref[...] = jax.nn.sigmoid(logits)
    logits_t = lax.dot_general(w, x, dn, preferred_element_type=jnp.float32)
    routing_t_ref[...] = jax.nn.sigmoid(logits_t) + bias_ref[...].reshape(N_EXPERTS, 1)


def _tc_scores(x_chunk, W, bias2d):
    grid = (TCHUNK // BT,)
    return pl.pallas_call(
        _tc_scores_block,
        grid=grid,
        in_specs=[
            pl.BlockSpec((BT, D_MODEL), lambda i: (i, 0)),
            pl.BlockSpec((N_EXPERTS, D_MODEL), lambda i: (0, 0)),
            pl.BlockSpec((1, N_EXPERTS), lambda i: (0, 0)),
        ],
        out_specs=[
            pl.BlockSpec((BT, N_EXPERTS), lambda i: (i, 0)),
            pl.BlockSpec((N_EXPERTS, BT), lambda i: (0, i)),
        ],
        out_shape=[
            jax.ShapeDtypeStruct((TCHUNK, N_EXPERTS), jnp.float32),
            jax.ShapeDtypeStruct((N_EXPERTS, TCHUNK), jnp.float32),
        ],
        compiler_params=pltpu.CompilerParams(
            dimension_semantics=("parallel",),
        ),
    )(x_chunk, W, bias2d)


def _splat_f32(v):
    return v + jnp.zeros((LANES,), jnp.float32)


def _splat_i32(v):
    return v + jnp.zeros((LANES,), jnp.int32)


def _sc_route_kernel(routing_t_hbm, bias_hbm, w_hbm, idx_hbm,
                     rt_vmem, bias_vmem, w_vmem, idx_vmem, row_vmem):
    wid = lax.axis_index("s") * NC + lax.axis_index("c")
    base = wid * TOK_PER_TILE

    pltpu.sync_copy(routing_t_hbm.at[:, pl.ds(base, TOK_PER_TILE)], rt_vmem)
    pltpu.sync_copy(bias_hbm, bias_vmem)

    lane_iota = lax.iota(jnp.int32, LANES)

    @pl.loop(0, CHUNKS_PER_TILE)
    def _chunk(ct):
        col = ct * LANES + lane_iota
        neg = _splat_f32(jnp.float32(NEG_INF))

        gs = []
        for g in range(N_GROUPS):
            e0 = g * GROUP_SIZE
            s = [rt_vmem[e0 + j, pl.ds(ct * LANES, LANES)] for j in range(GROUP_SIZE)]
            m1 = s[0]
            a1 = _splat_i32(jnp.int32(0))
            for j in range(1, GROUP_SIZE):
                c = s[j] > m1
                m1 = jnp.where(c, s[j], m1)
                a1 = jnp.where(c, _splat_i32(jnp.int32(j)), a1)
            m2 = neg
            for j in range(GROUP_SIZE):
                keep = a1 != _splat_i32(jnp.int32(j))
                m2 = jnp.maximum(m2, jnp.where(keep, s[j], neg))
            gs.append(m1 + m2)

        chosen = [lane_iota < 0 for _ in range(N_GROUPS)]
        picks = []
        for _ in range(TOPK_GROUPS):
            m = neg
            a = _splat_i32(jnp.int32(0))
            for g in range(N_GROUPS):
                c = jnp.logical_and(jnp.logical_not(chosen[g]), gs[g] > m)
                m = jnp.where(c, gs[g], m)
                a = jnp.where(c, _splat_i32(jnp.int32(g)), a)
            for g in range(N_GROUPS):
                chosen[g] = jnp.logical_or(chosen[g], a == _splat_i32(jnp.int32(g)))
            picks.append(a)

        def cmpx(i, j):
            lo = jnp.minimum(picks[i], picks[j])
            hi = jnp.maximum(picks[i], picks[j])
            picks[i] = lo
            picks[j] = hi
        cmpx(0, 2); cmpx(1, 3); cmpx(0, 1); cmpx(2, 3); cmpx(1, 2)

        cand = []
        rows = []
        for k in range(TOPK_GROUPS):
            rbase = picks[k] * GROUP_SIZE
            for j in range(GROUP_SIZE):
                r = rbase + _splat_i32(jnp.int32(j))
                rows.append(r)
                cand.append(plsc.load_gather(rt_vmem, [r, col]))

        for c in range(len(rows)):
            row_vmem[pl.ds(c * LANES, LANES)] = rows[c]

        wsum = _splat_f32(jnp.float32(0.0))
        wvals = []
        evals = []
        for _ in range(N_ACTIVATED):
            m = neg
            a = _splat_i32(jnp.int32(0))
            for c in range(len(cand)):
                sel = cand[c] > m
                m = jnp.where(sel, cand[c], m)
                a = jnp.where(sel, _splat_i32(jnp.int32(c)), a)
            for c in range(len(cand)):
                cand[c] = jnp.where(a == _splat_i32(jnp.int32(c)), neg, cand[c])
            e = plsc.load_gather(row_vmem, [a * LANES + lane_iota])
            bval = plsc.load_gather(bias_vmem, [e])
            w = m - bval
            wsum = wsum + w
            wvals.append(w)
            evals.append(e)

        scale = jnp.float32(ROUTE_SCALE) / jnp.maximum(wsum, jnp.float32(1e-9))
        for k in range(N_ACTIVATED):
            krow = _splat_i32(jnp.int32(k))
            plsc.store_scatter(w_vmem, [krow, col], wvals[k] * scale)
            plsc.store_scatter(idx_vmem, [krow, col], evals[k])

    pltpu.sync_copy(w_vmem, w_hbm.at[:, pl.ds(base, TOK_PER_TILE)])
    pltpu.sync_copy(idx_vmem, idx_hbm.at[:, pl.ds(base, TOK_PER_TILE)])


def _sc_route(routing_t, bias):
    mesh = plsc.VectorSubcoreMesh(core_axis_name="c", subcore_axis_name="s")
    cp = pltpu.CompilerParams()
    if "needs_layout_passes" in pltpu.CompilerParams.__dataclass_fields__:
        cp = dataclasses.replace(cp, needs_layout_passes=False)
    if "use_tc_tiling_on_sc" in pltpu.CompilerParams.__dataclass_fields__:
        cp = dataclasses.replace(cp, use_tc_tiling_on_sc=True)
    kern = pl.kernel(
        _sc_route_kernel,
        out_type=[
            jax.ShapeDtypeStruct((N_ACTIVATED, TCHUNK), jnp.float32),
            jax.ShapeDtypeStruct((N_ACTIVATED, TCHUNK), jnp.int32),
        ],
        mesh=mesh,
        scratch_types=[
            pltpu.VMEM((N_EXPERTS, TOK_PER_TILE), jnp.float32),
            pltpu.VMEM((N_EXPERTS,), jnp.float32),
            pltpu.VMEM((N_ACTIVATED, TOK_PER_TILE), jnp.float32),
            pltpu.VMEM((N_ACTIVATED, TOK_PER_TILE), jnp.int32),
            pltpu.VMEM((TOPK_GROUPS * GROUP_SIZE * LANES,), jnp.int32),
        ],
        compiler_params=cp,
    )
    return kern(routing_t, bias)


UBT = 2048


def _unflatten_block(wt_ref, it_ref, w_ref, i_ref):
    w_ref[...] = wt_ref[...].T
    i_ref[...] = it_ref[...].T


def _unflatten(w8t, idx8t):
    grid = (T // UBT,)
    return pl.pallas_call(
        _unflatten_block,
        grid=grid,
        in_specs=[
            pl.BlockSpec((N_ACTIVATED, UBT), lambda i: (0, i)),
            pl.BlockSpec((N_ACTIVATED, UBT), lambda i: (0, i)),
        ],
        out_specs=[
            pl.BlockSpec((UBT, N_ACTIVATED), lambda i: (i, 0)),
            pl.BlockSpec((UBT, N_ACTIVATED), lambda i: (i, 0)),
        ],
        out_shape=[
            jax.ShapeDtypeStruct((T, N_ACTIVATED), jnp.float32),
            jax.ShapeDtypeStruct((T, N_ACTIVATED), jnp.int32),
        ],
        compiler_params=pltpu.CompilerParams(
            dimension_semantics=("parallel",),
        ),
    )(w8t, idx8t)


def kernel(x, W, bias):
    bias2d = bias.reshape(1, N_EXPERTS)
    scores_parts = []
    w_parts = []
    idx_parts = []
    for c in range(N_CHUNKS):
        x_chunk = lax.slice_in_dim(x, c * TCHUNK, (c + 1) * TCHUNK, axis=0)
        scores_c, routing_t_c = _tc_scores(x_chunk, W, bias2d)
        w8t_c, idx8t_c = _sc_route(routing_t_c, bias)
        scores_parts.append(scores_c)
        w_parts.append(w8t_c)
        idx_parts.append(idx8t_c)
    if N_CHUNKS == 1:
        scores = scores_parts[0]
        weights = w_parts[0].T
        indices = idx_parts[0].T
    else:
        scores = jnp.concatenate(scores_parts, axis=0)
        weights = jnp.concatenate(w_parts, axis=1).T
        indices = jnp.concatenate(idx_parts, axis=1).T
    return (weights, indices.astype(jnp.int64), scores)


# speedup vs baseline: 1.8544x; 1.8544x over previous
import dataclasses
import functools

import jax
import jax.numpy as jnp
from jax import lax
from jax.experimental import pallas as pl
from jax.experimental.pallas import tpu as pltpu
from jax.experimental.pallas import tpu_sc as plsc

T = 16384
D_MODEL = 2048
N_EXPERTS = 64
N_ACTIVATED = 8
N_GROUPS = 8
GROUP_SIZE = N_EXPERTS // N_GROUPS
TOPK_GROUPS = 4
ROUTE_SCALE = 2.5

BT = 512

NEG_INF = float("-inf")

NC = 2
NS = 16
NW = NC * NS
LANES = 16

N_CHUNKS = 1
TCHUNK = T // N_CHUNKS
TOK_PER_TILE = TCHUNK // NW
CHUNKS_PER_TILE = TOK_PER_TILE // LANES


def _tc_scores_block(x_ref, w_ref, bias_ref, scores_out_ref, routing_t_ref):
    x = x_ref[...]
    w = w_ref[...]
    dn = (((1,), (1,)), ((), ()))
    logits = lax.dot_general(x, w, dn, preferred_element_type=jnp.float32)
    scores_out_ref[...] = jax.nn.sigmoid(logits)
    logits_t = lax.dot_general(w, x, dn, preferred_element_type=jnp.float32)
    routing_t_ref[...] = jax.nn.sigmoid(logits_t) + bias_ref[...].reshape(N_EXPERTS, 1)


def _tc_scores(x_chunk, W, bias2d):
    grid = (TCHUNK // BT,)
    return pl.pallas_call(
        _tc_scores_block,
        grid=grid,
        in_specs=[
            pl.BlockSpec((BT, D_MODEL), lambda i: (i, 0)),
            pl.BlockSpec((N_EXPERTS, D_MODEL), lambda i: (0, 0)),
            pl.BlockSpec((1, N_EXPERTS), lambda i: (0, 0)),
        ],
        out_specs=[
            pl.BlockSpec((BT, N_EXPERTS), lambda i: (i, 0)),
            pl.BlockSpec((N_EXPERTS, BT), lambda i: (0, i)),
        ],
        out_shape=[
            jax.ShapeDtypeStruct((TCHUNK, N_EXPERTS), jnp.float32),
            jax.ShapeDtypeStruct((N_EXPERTS, TCHUNK), jnp.float32),
        ],
        compiler_params=pltpu.CompilerParams(
            dimension_semantics=("parallel",),
        ),
    )(x_chunk, W, bias2d)


def _splat_f32(v):
    return v + jnp.zeros((LANES,), jnp.float32)


def _splat_i32(v):
    return v + jnp.zeros((LANES,), jnp.int32)


def _tree_argmax(vals, idxs):
    vs = list(vals)
    ix = list(idxs)
    while len(vs) > 1:
        nv, ni = [], []
        for i in range(0, len(vs) - 1, 2):
            take = vs[i + 1] > vs[i]
            nv.append(jnp.where(take, vs[i + 1], vs[i]))
            ni.append(jnp.where(take, ix[i + 1], ix[i]))
        if len(vs) % 2:
            nv.append(vs[-1])
            ni.append(ix[-1])
        vs, ix = nv, ni
    return vs[0], ix[0]


def _tree_max(vals):
    vs = list(vals)
    while len(vs) > 1:
        nv = [jnp.maximum(vs[i], vs[i + 1]) for i in range(0, len(vs) - 1, 2)]
        if len(vs) % 2:
            nv.append(vs[-1])
        vs = nv
    return vs[0]


def _sc_route_kernel(routing_t_hbm, bias_hbm, w_hbm, idx_hbm,
                     rt_vmem, bias_vmem, w_vmem, idx_vmem, row_vmem):
    wid = lax.axis_index("s") * NC + lax.axis_index("c")
    base = wid * TOK_PER_TILE

    pltpu.sync_copy(routing_t_hbm.at[:, pl.ds(base, TOK_PER_TILE)], rt_vmem)
    pltpu.sync_copy(bias_hbm, bias_vmem)

    lane_iota = lax.iota(jnp.int32, LANES)

    @pl.loop(0, CHUNKS_PER_TILE)
    def _chunk(ct):
        col = ct * LANES + lane_iota
        neg = _splat_f32(jnp.float32(NEG_INF))

        gs = []
        jidx = [_splat_i32(jnp.int32(j)) for j in range(GROUP_SIZE)]
        for g in range(N_GROUPS):
            e0 = g * GROUP_SIZE
            s = [rt_vmem[e0 + j, pl.ds(ct * LANES, LANES)] for j in range(GROUP_SIZE)]
            m1, a1 = _tree_argmax(s, jidx)
            m2 = _tree_max([jnp.where(a1 != jidx[j], s[j], neg) for j in range(GROUP_SIZE)])
            gs.append(m1 + m2)

        picks = []
        vals = list(gs)
        idxs = [_splat_i32(jnp.int32(g)) for g in range(N_GROUPS)]
        for _ in range(TOPK_GROUPS):
            m, a = _tree_argmax(vals, idxs)
            for g in range(N_GROUPS):
                vals[g] = jnp.where(a == idxs[g], neg, vals[g])
            picks.append(a)

        def cmpx(i, j):
            lo = jnp.minimum(picks[i], picks[j])
            hi = jnp.maximum(picks[i], picks[j])
            picks[i] = lo
            picks[j] = hi
        cmpx(0, 2); cmpx(1, 3); cmpx(0, 1); cmpx(2, 3); cmpx(1, 2)

        cand = []
        rows = []
        for k in range(TOPK_GROUPS):
            rbase = picks[k] * GROUP_SIZE
            for j in range(GROUP_SIZE):
                r = rbase + _splat_i32(jnp.int32(j))
                rows.append(r)
                cand.append(plsc.load_gather(rt_vmem, [r, col]))

        for c in range(len(rows)):
            row_vmem[pl.ds(c * LANES, LANES)] = rows[c]

        NCAND = TOPK_GROUPS * GROUP_SIZE
        cidx = [_splat_i32(jnp.int32(c)) for c in range(NCAND)]
        wsum = _splat_f32(jnp.float32(0.0))
        wvals = []
        evals = []
        for _ in range(N_ACTIVATED):
            m, a = _tree_argmax(cand, cidx)
            for c in range(NCAND):
                cand[c] = jnp.where(a == cidx[c], neg, cand[c])
            e = plsc.load_gather(row_vmem, [a * LANES + lane_iota])
            bval = plsc.load_gather(bias_vmem, [e])
            w = m - bval
            wsum = wsum + w
            wvals.append(w)
            evals.append(e)

        scale = jnp.float32(ROUTE_SCALE) / jnp.maximum(wsum, jnp.float32(1e-9))
        for k in range(N_ACTIVATED):
            krow = _splat_i32(jnp.int32(k))
            plsc.store_scatter(w_vmem, [krow, col], wvals[k] * scale)
            plsc.store_scatter(idx_vmem, [krow, col], evals[k])

    pltpu.sync_copy(w_vmem, w_hbm.at[:, pl.ds(base, TOK_PER_TILE)])
    pltpu.sync_copy(idx_vmem, idx_hbm.at[:, pl.ds(base, TOK_PER_TILE)])


def _sc_route(routing_t, bias):
    mesh = plsc.VectorSubcoreMesh(core_axis_name="c", subcore_axis_name="s")
    cp = pltpu.CompilerParams()
    if "needs_layout_passes" in pltpu.CompilerParams.__dataclass_fields__:
        cp = dataclasses.replace(cp, needs_layout_passes=False)
    if "use_tc_tiling_on_sc" in pltpu.CompilerParams.__dataclass_fields__:
        cp = dataclasses.replace(cp, use_tc_tiling_on_sc=True)
    kern = pl.kernel(
        _sc_route_kernel,
        out_type=[
            jax.ShapeDtypeStruct((N_ACTIVATED, TCHUNK), jnp.float32),
            jax.ShapeDtypeStruct((N_ACTIVATED, TCHUNK), jnp.int32),
        ],
        mesh=mesh,
        scratch_types=[
            pltpu.VMEM((N_EXPERTS, TOK_PER_TILE), jnp.float32),
            pltpu.VMEM((N_EXPERTS,), jnp.float32),
            pltpu.VMEM((N_ACTIVATED, TOK_PER_TILE), jnp.float32),
            pltpu.VMEM((N_ACTIVATED, TOK_PER_TILE), jnp.int32),
            pltpu.VMEM((TOPK_GROUPS * GROUP_SIZE * LANES,), jnp.int32),
        ],
        compiler_params=cp,
    )
    return kern(routing_t, bias)


UBT = 2048


def _unflatten_block(wt_ref, it_ref, w_ref, i_ref):
    w_ref[...] = wt_ref[...].T
    i_ref[...] = it_ref[...].T


def _unflatten(w8t, idx8t):
    grid = (T // UBT,)
    return pl.pallas_call(
        _unflatten_block,
        grid=grid,
        in_specs=[
            pl.BlockSpec((N_ACTIVATED, UBT), lambda i: (0, i)),
            pl.BlockSpec((N_ACTIVATED, UBT), lambda i: (0, i)),
        ],
        out_specs=[
            pl.BlockSpec((UBT, N_ACTIVATED), lambda i: (i, 0)),
            pl.BlockSpec((UBT, N_ACTIVATED), lambda i: (i, 0)),
        ],
        out_shape=[
            jax.ShapeDtypeStruct((T, N_ACTIVATED), jnp.float32),
            jax.ShapeDtypeStruct((T, N_ACTIVATED), jnp.int32),
        ],
        compiler_params=pltpu.CompilerParams(
            dimension_semantics=("parallel",),
        ),
    )(w8t, idx8t)


def kernel(x, W, bias):
    bias2d = bias.reshape(1, N_EXPERTS)
    scores_parts = []
    w_parts = []
    idx_parts = []
    for c in range(N_CHUNKS):
        x_chunk = lax.slice_in_dim(x, c * TCHUNK, (c + 1) * TCHUNK, axis=0)
        scores_c, routing_t_c = _tc_scores(x_chunk, W, bias2d)
        w8t_c, idx8t_c = _sc_route(routing_t_c, bias)
        scores_parts.append(scores_c)
        w_parts.append(w8t_c)
        idx_parts.append(idx8t_c)
    if N_CHUNKS == 1:
        scores = scores_parts[0]
        weights = w_parts[0].T
        indices = idx_parts[0].T
    else:
        scores = jnp.concatenate(scores_parts, axis=0)
        weights = jnp.concatenate(w_parts, axis=1).T
        indices = jnp.concatenate(idx_parts, axis=1).T
    return (weights, indices.astype(jnp.int64), scores)


# speedup vs baseline: 2.0422x; 1.1012x over previous
import dataclasses
import functools

import jax
import jax.numpy as jnp
from jax import lax
from jax.experimental import pallas as pl
from jax.experimental.pallas import tpu as pltpu
from jax.experimental.pallas import tpu_sc as plsc

T = 16384
D_MODEL = 2048
N_EXPERTS = 64
N_ACTIVATED = 8
N_GROUPS = 8
GROUP_SIZE = N_EXPERTS // N_GROUPS
TOPK_GROUPS = 4
ROUTE_SCALE = 2.5

BT = 1024

NEG_INF = float("-inf")

NC = 2
NS = 16
NW = NC * NS
LANES = 16

N_CHUNKS = 1
TCHUNK = T // N_CHUNKS
TOK_PER_TILE = TCHUNK // NW
CHUNKS_PER_TILE = TOK_PER_TILE // LANES


def _tc_scores_block(x_ref, w_ref, bias_ref, scores_out_ref, routing_t_ref):
    x = x_ref[...]
    w = w_ref[...]
    dn = (((1,), (1,)), ((), ()))
    logits = lax.dot_general(x, w, dn, preferred_element_type=jnp.float32)
    scores_out_ref[...] = jax.nn.sigmoid(logits)
    logits_t = lax.dot_general(w, x, dn, preferred_element_type=jnp.float32)
    routing_t_ref[...] = jax.nn.sigmoid(logits_t) + bias_ref[...].reshape(N_EXPERTS, 1)


def _tc_scores(x_chunk, W, bias2d):
    grid = (TCHUNK // BT,)
    return pl.pallas_call(
        _tc_scores_block,
        grid=grid,
        in_specs=[
            pl.BlockSpec((BT, D_MODEL), lambda i: (i, 0)),
            pl.BlockSpec((N_EXPERTS, D_MODEL), lambda i: (0, 0)),
            pl.BlockSpec((1, N_EXPERTS), lambda i: (0, 0)),
        ],
        out_specs=[
            pl.BlockSpec((BT, N_EXPERTS), lambda i: (i, 0)),
            pl.BlockSpec((N_EXPERTS, BT), lambda i: (0, i)),
        ],
        out_shape=[
            jax.ShapeDtypeStruct((TCHUNK, N_EXPERTS), jnp.float32),
            jax.ShapeDtypeStruct((N_EXPERTS, TCHUNK), jnp.float32),
        ],
        compiler_params=pltpu.CompilerParams(
            dimension_semantics=("parallel",),
        ),
    )(x_chunk, W, bias2d)


def _splat_f32(v):
    return v + jnp.zeros((LANES,), jnp.float32)


def _splat_i32(v):
    return v + jnp.zeros((LANES,), jnp.int32)


def _tree_argmax(vals, idxs):
    vs = list(vals)
    ix = list(idxs)
    while len(vs) > 1:
        nv, ni = [], []
        for i in range(0, len(vs) - 1, 2):
            take = vs[i + 1] > vs[i]
            nv.append(jnp.where(take, vs[i + 1], vs[i]))
            ni.append(jnp.where(take, ix[i + 1], ix[i]))
        if len(vs) % 2:
            nv.append(vs[-1])
            ni.append(ix[-1])
        vs, ix = nv, ni
    return vs[0], ix[0]


def _tree_max(vals):
    vs = list(vals)
    while len(vs) > 1:
        nv = [jnp.maximum(vs[i], vs[i + 1]) for i in range(0, len(vs) - 1, 2)]
        if len(vs) % 2:
            nv.append(vs[-1])
        vs = nv
    return vs[0]


def _sc_route_kernel(routing_t_hbm, bias_hbm, w_hbm, idx_hbm,
                     rt_vmem, bias_vmem, w_vmem, idx_vmem, row_vmem):
    wid = lax.axis_index("s") * NC + lax.axis_index("c")
    base = wid * TOK_PER_TILE

    pltpu.sync_copy(routing_t_hbm.at[:, pl.ds(base, TOK_PER_TILE)], rt_vmem)
    pltpu.sync_copy(bias_hbm, bias_vmem)

    lane_iota = lax.iota(jnp.int32, LANES)

    @pl.loop(0, CHUNKS_PER_TILE)
    def _chunk(ct):
        col = ct * LANES + lane_iota
        neg = _splat_f32(jnp.float32(NEG_INF))

        gs = []
        jidx = [_splat_i32(jnp.int32(j)) for j in range(GROUP_SIZE)]
        for g in range(N_GROUPS):
            e0 = g * GROUP_SIZE
            s = [rt_vmem[e0 + j, pl.ds(ct * LANES, LANES)] for j in range(GROUP_SIZE)]
            m1, a1 = _tree_argmax(s, jidx)
            m2 = _tree_max([jnp.where(a1 != jidx[j], s[j], neg) for j in range(GROUP_SIZE)])
            gs.append(m1 + m2)

        picks = []
        vals = list(gs)
        idxs = [_splat_i32(jnp.int32(g)) for g in range(N_GROUPS)]
        for _ in range(TOPK_GROUPS):
            m, a = _tree_argmax(vals, idxs)
            for g in range(N_GROUPS):
                vals[g] = jnp.where(a == idxs[g], neg, vals[g])
            picks.append(a)

        def cmpx(i, j):
            lo = jnp.minimum(picks[i], picks[j])
            hi = jnp.maximum(picks[i], picks[j])
            picks[i] = lo
            picks[j] = hi
        cmpx(0, 2); cmpx(1, 3); cmpx(0, 1); cmpx(2, 3); cmpx(1, 2)

        cand = []
        rows = []
        for k in range(TOPK_GROUPS):
            rbase = picks[k] * GROUP_SIZE
            for j in range(GROUP_SIZE):
                r = rbase + _splat_i32(jnp.int32(j))
                rows.append(r)
                cand.append(plsc.load_gather(rt_vmem, [r, col]))

        for c in range(len(rows)):
            row_vmem[pl.ds(c * LANES, LANES)] = rows[c]

        NCAND = TOPK_GROUPS * GROUP_SIZE
        cidx = [_splat_i32(jnp.int32(c)) for c in range(NCAND)]
        wsum = _splat_f32(jnp.float32(0.0))
        wvals = []
        evals = []
        for _ in range(N_ACTIVATED):
            m, a = _tree_argmax(cand, cidx)
            for c in range(NCAND):
                cand[c] = jnp.where(a == cidx[c], neg, cand[c])
            e = plsc.load_gather(row_vmem, [a * LANES + lane_iota])
            bval = plsc.load_gather(bias_vmem, [e])
            w = m - bval
            wsum = wsum + w
            wvals.append(w)
            evals.append(e)

        scale = jnp.float32(ROUTE_SCALE) / jnp.maximum(wsum, jnp.float32(1e-9))
        for k in range(N_ACTIVATED):
            krow = _splat_i32(jnp.int32(k))
            plsc.store_scatter(w_vmem, [krow, col], wvals[k] * scale)
            plsc.store_scatter(idx_vmem, [krow, col], evals[k])

    pltpu.sync_copy(w_vmem, w_hbm.at[:, pl.ds(base, TOK_PER_TILE)])
    pltpu.sync_copy(idx_vmem, idx_hbm.at[:, pl.ds(base, TOK_PER_TILE)])


def _sc_route(routing_t, bias):
    mesh = plsc.VectorSubcoreMesh(core_axis_name="c", subcore_axis_name="s")
    cp = pltpu.CompilerParams()
    if "needs_layout_passes" in pltpu.CompilerParams.__dataclass_fields__:
        cp = dataclasses.replace(cp, needs_layout_passes=False)
    if "use_tc_tiling_on_sc" in pltpu.CompilerParams.__dataclass_fields__:
        cp = dataclasses.replace(cp, use_tc_tiling_on_sc=True)
    kern = pl.kernel(
        _sc_route_kernel,
        out_type=[
            jax.ShapeDtypeStruct((N_ACTIVATED, TCHUNK), jnp.float32),
            jax.ShapeDtypeStruct((N_ACTIVATED, TCHUNK), jnp.int32),
        ],
        mesh=mesh,
        scratch_types=[
            pltpu.VMEM((N_EXPERTS, TOK_PER_TILE), jnp.float32),
            pltpu.VMEM((N_EXPERTS,), jnp.float32),
            pltpu.VMEM((N_ACTIVATED, TOK_PER_TILE), jnp.float32),
            pltpu.VMEM((N_ACTIVATED, TOK_PER_TILE), jnp.int32),
            pltpu.VMEM((TOPK_GROUPS * GROUP_SIZE * LANES,), jnp.int32),
        ],
        compiler_params=cp,
    )
    return kern(routing_t, bias)


UBT = 2048


def _unflatten_block(wt_ref, it_ref, w_ref, i_ref):
    w_ref[...] = wt_ref[...].T
    i_ref[...] = it_ref[...].T


def _unflatten(w8t, idx8t):
    grid = (T // UBT,)
    return pl.pallas_call(
        _unflatten_block,
        grid=grid,
        in_specs=[
            pl.BlockSpec((N_ACTIVATED, UBT), lambda i: (0, i)),
            pl.BlockSpec((N_ACTIVATED, UBT), lambda i: (0, i)),
        ],
        out_specs=[
            pl.BlockSpec((UBT, N_ACTIVATED), lambda i: (i, 0)),
            pl.BlockSpec((UBT, N_ACTIVATED), lambda i: (i, 0)),
        ],
        out_shape=[
            jax.ShapeDtypeStruct((T, N_ACTIVATED), jnp.float32),
            jax.ShapeDtypeStruct((T, N_ACTIVATED), jnp.int32),
        ],
        compiler_params=pltpu.CompilerParams(
            dimension_semantics=("parallel",),
        ),
    )(w8t, idx8t)


def kernel(x, W, bias):
    bias2d = bias.reshape(1, N_EXPERTS)
    scores_parts = []
    w_parts = []
    idx_parts = []
    for c in range(N_CHUNKS):
        x_chunk = lax.slice_in_dim(x, c * TCHUNK, (c + 1) * TCHUNK, axis=0)
        scores_c, routing_t_c = _tc_scores(x_chunk, W, bias2d)
        w8t_c, idx8t_c = _sc_route(routing_t_c, bias)
        scores_parts.append(scores_c)
        w_parts.append(w8t_c)
        idx_parts.append(idx8t_c)
    if N_CHUNKS == 1:
        scores = scores_parts[0]
        weights = w_parts[0].T
        indices = idx_parts[0].T
    else:
        scores = jnp.concatenate(scores_parts, axis=0)
        weights = jnp.concatenate(w_parts, axis=1).T
        indices = jnp.concatenate(idx_parts, axis=1).T
    return (weights, indices.astype(jnp.int64), scores)


# speedup vs baseline: 2.1201x; 1.0382x over previous
import dataclasses
import functools

import jax
import jax.numpy as jnp
from jax import lax
from jax.experimental import pallas as pl
from jax.experimental.pallas import tpu as pltpu
from jax.experimental.pallas import tpu_sc as plsc

T = 16384
D_MODEL = 2048
N_EXPERTS = 64
N_ACTIVATED = 8
N_GROUPS = 8
GROUP_SIZE = N_EXPERTS // N_GROUPS
TOPK_GROUPS = 4
ROUTE_SCALE = 2.5

BT = 2048

NEG_INF = float("-inf")

NC = 2
NS = 16
NW = NC * NS
LANES = 16

N_CHUNKS = 1
TCHUNK = T // N_CHUNKS
TOK_PER_TILE = TCHUNK // NW
CHUNKS_PER_TILE = TOK_PER_TILE // LANES


def _tc_scores_block(x_ref, w_ref, bias_ref, scores_out_ref, routing_t_ref):
    x = x_ref[...]
    w = w_ref[...]
    dn = (((1,), (1,)), ((), ()))
    logits = lax.dot_general(x, w, dn, preferred_element_type=jnp.float32)
    scores_out_ref[...] = jax.nn.sigmoid(logits)
    logits_t = lax.dot_general(w, x, dn, preferred_element_type=jnp.float32)
    routing_t_ref[...] = jax.nn.sigmoid(logits_t) + bias_ref[...].reshape(N_EXPERTS, 1)


def _tc_scores(x_chunk, W, bias2d):
    grid = (TCHUNK // BT,)
    return pl.pallas_call(
        _tc_scores_block,
        grid=grid,
        in_specs=[
            pl.BlockSpec((BT, D_MODEL), lambda i: (i, 0)),
            pl.BlockSpec((N_EXPERTS, D_MODEL), lambda i: (0, 0)),
            pl.BlockSpec((1, N_EXPERTS), lambda i: (0, 0)),
        ],
        out_specs=[
            pl.BlockSpec((BT, N_EXPERTS), lambda i: (i, 0)),
            pl.BlockSpec((N_EXPERTS, BT), lambda i: (0, i)),
        ],
        out_shape=[
            jax.ShapeDtypeStruct((TCHUNK, N_EXPERTS), jnp.float32),
            jax.ShapeDtypeStruct((N_EXPERTS, TCHUNK), jnp.float32),
        ],
        compiler_params=pltpu.CompilerParams(
            dimension_semantics=("parallel",),
        ),
    )(x_chunk, W, bias2d)


def _splat_f32(v):
    return v + jnp.zeros((LANES,), jnp.float32)


def _splat_i32(v):
    return v + jnp.zeros((LANES,), jnp.int32)


def _tree_argmax(vals, idxs):
    vs = list(vals)
    ix = list(idxs)
    while len(vs) > 1:
        nv, ni = [], []
        for i in range(0, len(vs) - 1, 2):
            take = vs[i + 1] > vs[i]
            nv.append(jnp.where(take, vs[i + 1], vs[i]))
            ni.append(jnp.where(take, ix[i + 1], ix[i]))
        if len(vs) % 2:
            nv.append(vs[-1])
            ni.append(ix[-1])
        vs, ix = nv, ni
    return vs[0], ix[0]


def _tree_max(vals):
    vs = list(vals)
    while len(vs) > 1:
        nv = [jnp.maximum(vs[i], vs[i + 1]) for i in range(0, len(vs) - 1, 2)]
        if len(vs) % 2:
            nv.append(vs[-1])
        vs = nv
    return vs[0]


def _sc_route_kernel(routing_t_hbm, bias_hbm, w_hbm, idx_hbm,
                     rt_vmem, bias_vmem, w_vmem, idx_vmem, row_vmem):
    wid = lax.axis_index("s") * NC + lax.axis_index("c")
    base = wid * TOK_PER_TILE

    pltpu.sync_copy(routing_t_hbm.at[:, pl.ds(base, TOK_PER_TILE)], rt_vmem)
    pltpu.sync_copy(bias_hbm, bias_vmem)

    lane_iota = lax.iota(jnp.int32, LANES)

    @pl.loop(0, CHUNKS_PER_TILE)
    def _chunk(ct):
        col = ct * LANES + lane_iota
        neg = _splat_f32(jnp.float32(NEG_INF))

        gs = []
        jidx = [_splat_i32(jnp.int32(j)) for j in range(GROUP_SIZE)]
        for g in range(N_GROUPS):
            e0 = g * GROUP_SIZE
            s = [rt_vmem[e0 + j, pl.ds(ct * LANES, LANES)] for j in range(GROUP_SIZE)]
            m1, a1 = _tree_argmax(s, jidx)
            m2 = _tree_max([jnp.where(a1 != jidx[j], s[j], neg) for j in range(GROUP_SIZE)])
            gs.append(m1 + m2)

        picks = []
        vals = list(gs)
        idxs = [_splat_i32(jnp.int32(g)) for g in range(N_GROUPS)]
        for _ in range(TOPK_GROUPS):
            m, a = _tree_argmax(vals, idxs)
            for g in range(N_GROUPS):
                vals[g] = jnp.where(a == idxs[g], neg, vals[g])
            picks.append(a)

        def cmpx(i, j):
            lo = jnp.minimum(picks[i], picks[j])
            hi = jnp.maximum(picks[i], picks[j])
            picks[i] = lo
            picks[j] = hi
        cmpx(0, 2); cmpx(1, 3); cmpx(0, 1); cmpx(2, 3); cmpx(1, 2)

        cand = []
        rows = []
        for k in range(TOPK_GROUPS):
            rbase = picks[k] * GROUP_SIZE
            for j in range(GROUP_SIZE):
                r = rbase + _splat_i32(jnp.int32(j))
                rows.append(r)
                cand.append(plsc.load_gather(rt_vmem, [r, col]))

        for c in range(len(rows)):
            row_vmem[pl.ds(c * LANES, LANES)] = rows[c]

        NCAND = TOPK_GROUPS * GROUP_SIZE
        cidx = [_splat_i32(jnp.int32(c)) for c in range(NCAND)]
        wsum = _splat_f32(jnp.float32(0.0))
        wvals = []
        evals = []
        for _ in range(N_ACTIVATED):
            m, a = _tree_argmax(cand, cidx)
            for c in range(NCAND):
                cand[c] = jnp.where(a == cidx[c], neg, cand[c])
            e = plsc.load_gather(row_vmem, [a * LANES + lane_iota])
            bval = plsc.load_gather(bias_vmem, [e])
            w = m - bval
            wsum = wsum + w
            wvals.append(w)
            evals.append(e)

        scale = jnp.float32(ROUTE_SCALE) / jnp.maximum(wsum, jnp.float32(1e-9))
        for k in range(N_ACTIVATED):
            krow = _splat_i32(jnp.int32(k))
            plsc.store_scatter(w_vmem, [krow, col], wvals[k] * scale)
            plsc.store_scatter(idx_vmem, [krow, col], evals[k])

    pltpu.sync_copy(w_vmem, w_hbm.at[:, pl.ds(base, TOK_PER_TILE)])
    pltpu.sync_copy(idx_vmem, idx_hbm.at[:, pl.ds(base, TOK_PER_TILE)])


def _sc_route(routing_t, bias):
    mesh = plsc.VectorSubcoreMesh(core_axis_name="c", subcore_axis_name="s")
    cp = pltpu.CompilerParams()
    if "needs_layout_passes" in pltpu.CompilerParams.__dataclass_fields__:
        cp = dataclasses.replace(cp, needs_layout_passes=False)
    if "use_tc_tiling_on_sc" in pltpu.CompilerParams.__dataclass_fields__:
        cp = dataclasses.replace(cp, use_tc_tiling_on_sc=True)
    kern = pl.kernel(
        _sc_route_kernel,
        out_type=[
            jax.ShapeDtypeStruct((N_ACTIVATED, TCHUNK), jnp.float32),
            jax.ShapeDtypeStruct((N_ACTIVATED, TCHUNK), jnp.int32),
        ],
        mesh=mesh,
        scratch_types=[
            pltpu.VMEM((N_EXPERTS, TOK_PER_TILE), jnp.float32),
            pltpu.VMEM((N_EXPERTS,), jnp.float32),
            pltpu.VMEM((N_ACTIVATED, TOK_PER_TILE), jnp.float32),
            pltpu.VMEM((N_ACTIVATED, TOK_PER_TILE), jnp.int32),
            pltpu.VMEM((TOPK_GROUPS * GROUP_SIZE * LANES,), jnp.int32),
        ],
        compiler_params=cp,
    )
    return kern(routing_t, bias)


UBT = 2048


def _unflatten_block(wt_ref, it_ref, w_ref, i_ref):
    w_ref[...] = wt_ref[...].T
    i_ref[...] = it_ref[...].T


def _unflatten(w8t, idx8t):
    grid = (T // UBT,)
    return pl.pallas_call(
        _unflatten_block,
        grid=grid,
        in_specs=[
            pl.BlockSpec((N_ACTIVATED, UBT), lambda i: (0, i)),
            pl.BlockSpec((N_ACTIVATED, UBT), lambda i: (0, i)),
        ],
        out_specs=[
            pl.BlockSpec((UBT, N_ACTIVATED), lambda i: (i, 0)),
            pl.BlockSpec((UBT, N_ACTIVATED), lambda i: (i, 0)),
        ],
        out_shape=[
            jax.ShapeDtypeStruct((T, N_ACTIVATED), jnp.float32),
            jax.ShapeDtypeStruct((T, N_ACTIVATED), jnp.int32),
        ],
        compiler_params=pltpu.CompilerParams(
            dimension_semantics=("parallel",),
        ),
    )(w8t, idx8t)


def kernel(x, W, bias):
    bias2d = bias.reshape(1, N_EXPERTS)
    scores_parts = []
    w_parts = []
    idx_parts = []
    for c in range(N_CHUNKS):
        x_chunk = lax.slice_in_dim(x, c * TCHUNK, (c + 1) * TCHUNK, axis=0)
        scores_c, routing_t_c = _tc_scores(x_chunk, W, bias2d)
        w8t_c, idx8t_c = _sc_route(routing_t_c, bias)
        scores_parts.append(scores_c)
        w_parts.append(w8t_c)
        idx_parts.append(idx8t_c)
    if N_CHUNKS == 1:
        scores = scores_parts[0]
        weights = w_parts[0].T
        indices = idx_parts[0].T
    else:
        scores = jnp.concatenate(scores_parts, axis=0)
        weights = jnp.concatenate(w_parts, axis=1).T
        indices = jnp.concatenate(idx_parts, axis=1).T
    return (weights, indices.astype(jnp.int64), scores)
